# baseline (speedup 1.0000x reference)
import functools

import jax
import jax.numpy as jnp
from jax import lax
from jax.experimental import pallas as pl
from jax.experimental.pallas import tpu as pltpu
from jax.experimental.pallas import tpu_sc as plsc

NV = 6890
NN = 689
K = 3
NB = 18

NWORK = 32
VPT = 224
VP = NWORK * VPT
WGRP = VPT // 16
AGRP = 2
NP = 704
NP3 = 3 * NP
VP3 = 3 * VP
RINGP = NWORK * AGRP * 16 * NB

_mesh = plsc.VectorSubcoreMesh(core_axis_name="c", subcore_axis_name="s")
_sc_params = pltpu.CompilerParams(needs_layout_passes=False)

_SIN_C = (1.0, -1.0 / 6, 1.0 / 120, -1.0 / 5040, 1.0 / 362880,
          -1.0 / 39916800)
_COS_C = (1.0, -1.0 / 2, 1.0 / 24, -1.0 / 720, 1.0 / 40320, -1.0 / 3628800,
          1.0 / 479001600)


def _poly(q, coeffs):
    acc = jnp.full((16,), coeffs[-1], jnp.float32)
    for c in coeffs[-2::-1]:
        acc = acc * q + c
    return acc


@functools.partial(
    pl.kernel,
    mesh=_mesh,
    out_type=(
        jax.ShapeDtypeStruct((VP3,), jnp.float32),
        jax.ShapeDtypeStruct((NWORK * 16,), jnp.float32),
    ),
    compiler_params=_sc_params,
    scratch_types=[
        pltpu.VMEM((VP3,), jnp.float32),
        pltpu.VMEM((VPT * 3,), jnp.float32),
        pltpu.VMEM((VPT * 3,), jnp.int32),
        pltpu.VMEM((NP3,), jnp.float32),
        pltpu.VMEM((NP3,), jnp.float32),
        pltpu.VMEM((NP,), jnp.int32),
        pltpu.VMEM((AGRP * 16 * NB,), jnp.int32),
        pltpu.VMEM((18 * NP,), jnp.float32),
        pltpu.VMEM((VPT * 3,), jnp.float32),
        pltpu.VMEM((16,), jnp.float32),
        pltpu.SemaphoreType.DMA,
    ],
)
def _dgraph(v_hbm, w_hbm, ix_hbm, rv_hbm, tv_hbm, nidx_hbm, ring_hbm,
            warp_hbm, loss_hbm,
            v_v, w_v, ix_v, rv_v, tv_v, nidx_v, ring_v, tab_v, out_v,
            loss_v, sem):
    wid = lax.axis_index("s") * 2 + lax.axis_index("c")
    base = wid * VPT

    cps = [
        pltpu.async_copy(v_hbm, v_v, sem),
        pltpu.async_copy(w_hbm.at[pl.ds(base * 3, VPT * 3)], w_v, sem),
        pltpu.async_copy(ix_hbm.at[pl.ds(base * 3, VPT * 3)], ix_v, sem),
        pltpu.async_copy(rv_hbm, rv_v, sem),
        pltpu.async_copy(tv_hbm, tv_v, sem),
        pltpu.async_copy(nidx_hbm, nidx_v, sem),
        pltpu.async_copy(
            ring_hbm.at[pl.ds(wid * AGRP * 16 * NB, AGRP * 16 * NB)],
            ring_v, sem),
    ]
    for cp in cps:
        cp.wait()

    ids = lax.iota(jnp.int32, 16)

    def build_group(g, carry):
        nids = g * 16 + ids
        n3 = nids * 3
        rx = plsc.load_gather(rv_v, [n3])
        ry = plsc.load_gather(rv_v, [n3 + 1])
        rz = plsc.load_gather(rv_v, [n3 + 2])
        xa = rx + 1e-8
        ya = ry + 1e-8
        za = rz + 1e-8
        ss = xa * xa + ya * ya + za * za
        u = plsc.bitcast(ss, jnp.int32)
        u = 0x5F3759DF - lax.shift_right_logical(u, 1)
        y = plsc.bitcast(u, jnp.float32)
        for _ in range(4):
            y = y * (1.5 - 0.5 * ss * y * y)
        ang = ss * y
        ax = rx / ang
        ay = ry / ang
        az = rz / ang
        q = ang * ang
        s = ang * _poly(q, _SIN_C)
        c = _poly(q, _COS_C)
        cc = 1.0 - c
        r00 = c + cc * ax * ax
        r01 = cc * ax * ay - s * az
        r02 = cc * ax * az + s * ay
        r10 = cc * ax * ay + s * az
        r11 = c + cc * ay * ay
        r12 = cc * ay * az - s * ax
        r20 = cc * ax * az - s * ay
        r21 = cc * ay * az + s * ax
        r22 = c + cc * az * az
        j = plsc.load_gather(nidx_v, [nids])
        j3 = j * 3
        nx = plsc.load_gather(v_v, [j3])
        ny = plsc.load_gather(v_v, [j3 + 1])
        nz = plsc.load_gather(v_v, [j3 + 2])
        tx = plsc.load_gather(tv_v, [n3])
        ty = plsc.load_gather(tv_v, [n3 + 1])
        tz = plsc.load_gather(tv_v, [n3 + 2])
        pmx = nx + tx
        pmy = ny + ty
        pmz = nz + tz
        sl = pl.ds(g * 16, 16)
        tab_v[sl] = r00
        tab_v[pl.ds(NP + g * 16, 16)] = r01
        tab_v[pl.ds(2 * NP + g * 16, 16)] = r02
        tab_v[pl.ds(3 * NP + g * 16, 16)] = r10
        tab_v[pl.ds(4 * NP + g * 16, 16)] = r11
        tab_v[pl.ds(5 * NP + g * 16, 16)] = r12
        tab_v[pl.ds(6 * NP + g * 16, 16)] = r20
        tab_v[pl.ds(7 * NP + g * 16, 16)] = r21
        tab_v[pl.ds(8 * NP + g * 16, 16)] = r22
        tab_v[pl.ds(9 * NP + g * 16, 16)] = pmx - (
            r00 * nx + r01 * ny + r02 * nz)
        tab_v[pl.ds(10 * NP + g * 16, 16)] = pmy - (
            r10 * nx + r11 * ny + r12 * nz)
        tab_v[pl.ds(11 * NP + g * 16, 16)] = pmz - (
            r20 * nx + r21 * ny + r22 * nz)
        tab_v[pl.ds(12 * NP + g * 16, 16)] = pmx
        tab_v[pl.ds(13 * NP + g * 16, 16)] = pmy
        tab_v[pl.ds(14 * NP + g * 16, 16)] = pmz
        tab_v[pl.ds(15 * NP + g * 16, 16)] = nx
        tab_v[pl.ds(16 * NP + g * 16, 16)] = ny
        tab_v[pl.ds(17 * NP + g * 16, 16)] = nz
        return carry

    lax.fori_loop(0, NP // 16, build_group, 0)

    def warp_group(g, carry):
        l3 = (g * 16 + ids) * 3
        v3 = base * 3 + l3
        vx = plsc.load_gather(v_v, [v3])
        vy = plsc.load_gather(v_v, [v3 + 1])
        vz = plsc.load_gather(v_v, [v3 + 2])
        acc = [jnp.zeros((16,), jnp.float32) for _ in range(12)]
        for k in range(K):
            j = plsc.load_gather(ix_v, [l3 + k])
            w = plsc.load_gather(w_v, [l3 + k])
            for t in range(12):
                acc[t] = acc[t] + w * plsc.load_gather(tab_v, [j + t * NP])
        plsc.store_scatter(out_v, [l3],
                           acc[0] * vx + acc[1] * vy + acc[2] * vz + acc[9])
        plsc.store_scatter(out_v, [l3 + 1],
                           acc[3] * vx + acc[4] * vy + acc[5] * vz + acc[10])
        plsc.store_scatter(out_v, [l3 + 2],
                           acc[6] * vx + acc[7] * vy + acc[8] * vz + acc[11])
        return carry

    lax.fori_loop(0, WGRP, warp_group, 0)
    pltpu.async_copy(out_v, warp_hbm.at[pl.ds(base * 3, VPT * 3)], sem).wait()

    acc_loss = jnp.zeros((16,), jnp.float32)
    for gg in range(AGRP):
        gbase = (wid * AGRP + gg) * 16
        gclamp = jnp.minimum(gbase, NP - 16)
        r = [tab_v[pl.ds(t * NP + gclamp, 16)] for t in range(9)]
        pm = [tab_v[pl.ds((12 + ci) * NP + gclamp, 16)] for ci in range(3)]
        nn = [tab_v[pl.ds((15 + ci) * NP + gclamp, 16)] for ci in range(3)]
        valid = (gbase + ids) < NN
        for h in range(NB):
            m = plsc.load_gather(ring_v, [(gg * 16 + ids) * NB + h])
            nm = [plsc.load_gather(tab_v, [m + (15 + ci) * NP])
                  for ci in range(3)]
            pmm = [plsc.load_gather(tab_v, [m + (12 + ci) * NP])
                   for ci in range(3)]
            dx = nn[0] - nm[0]
            dy = nn[1] - nm[1]
            dz = nn[2] - nm[2]
            ex = pm[0] - pmm[0] - (r[0] * dx + r[1] * dy + r[2] * dz)
            ey = pm[1] - pmm[1] - (r[3] * dx + r[4] * dy + r[5] * dz)
            ez = pm[2] - pmm[2] - (r[6] * dx + r[7] * dy + r[8] * dz)
            e2 = ex * ex + ey * ey + ez * ez
            acc_loss = acc_loss + jnp.where(valid, e2, 0.0)
    loss_v[...] = acc_loss
    pltpu.async_copy(loss_v, loss_hbm.at[pl.ds(wid * 16, 16)], sem).wait()


def _padto(x, n):
    return jnp.concatenate([x, jnp.zeros((n - x.shape[0],), x.dtype)])


def kernel(vertices, opt_d_rotations, opt_d_translations, weights, nodes_idx,
           influence_nodes_idx, one_ring_neigh):
    i32 = jnp.int32
    vflat = _padto(vertices.reshape(-1), VP3)
    wflat = _padto(weights.reshape(-1), VP3)
    ixflat = _padto(influence_nodes_idx.astype(i32).reshape(-1), VP3)
    rvflat = _padto(opt_d_rotations.reshape(-1), NP3)
    tvflat = _padto(opt_d_translations.reshape(-1), NP3)
    nidx = _padto(nodes_idx.astype(i32), NP)
    ringflat = _padto(one_ring_neigh.astype(i32).reshape(-1), RINGP)

    warp, loss_part = _dgraph(vflat, wflat, ixflat, rvflat, tvflat, nidx,
                              ringflat)
    warped = warp[:NV * 3].reshape(1, NV, 3)
    arap = jnp.sum(loss_part) / jnp.float32(NN)
    return warped, arap
